# baseline (speedup 1.0000x reference)
import functools

import jax
import jax.numpy as jnp
from jax import lax
from jax.experimental import pallas as pl
from jax.experimental.pallas import tpu as pltpu
from jax.experimental.pallas import tpu_sc as plsc

NC = 2
NS = 16
NW = NC * NS
EB = 128
F0 = 0.32

R = 512


def _sc_mesh():
  return plsc.VectorSubcoreMesh(
      core_axis_name="c", subcore_axis_name="s",
      num_cores=NC, num_subcores=NS)


def _make_deg_kernel(n_pad, nb):
  chunk = n_pad // NS

  @functools.partial(
      pl.kernel,
      mesh=_sc_mesh(),
      out_type=jax.ShapeDtypeStruct((NC, n_pad), jnp.float32),
      scratch_types=[
          pltpu.VMEM((nb, EB), jnp.int32),
          pltpu.VMEM((EB,), jnp.float32),
          pltpu.VMEM_SHARED((n_pad,), jnp.float32),
      ],
  )
  def deg_kernel(dst_hbm, ones_hbm, zeros_hbm, out_hbm, dst_v, ones_v, shared):
    c = lax.axis_index("c")
    s = lax.axis_index("s")
    wid = s * NC + c
    pltpu.sync_copy(zeros_hbm.at[pl.ds(s * chunk, chunk)],
                    shared.at[pl.ds(s * chunk, chunk)])
    pltpu.sync_copy(dst_hbm.at[wid], dst_v)
    pltpu.sync_copy(ones_hbm, ones_v)
    plsc.subcore_barrier()

    def body(j, carry):
      pltpu.sync_copy(ones_v, shared.at[dst_v.at[j]], add=True)
      return carry

    lax.fori_loop(0, nb, body, 0)
    plsc.subcore_barrier()
    pltpu.sync_copy(shared.at[pl.ds(s * chunk, chunk)],
                    out_hbm.at[c, pl.ds(s * chunk, chunk)])

  return deg_kernel


def _make_seg_kernel(n_pad, nb_max, nb0, nb1, d):
  chunk = n_pad // NS

  @functools.partial(
      pl.kernel,
      mesh=_sc_mesh(),
      out_type=jax.ShapeDtypeStruct((NC, n_pad, d), jnp.float32),
      scratch_types=[
          pltpu.VMEM((nb_max, EB), jnp.int32),
          pltpu.VMEM((nb_max, EB), jnp.int32),
          pltpu.VMEM((EB, d), jnp.float32),
          pltpu.VMEM_SHARED((n_pad, d), jnp.float32),
          pltpu.SemaphoreType.DMA,
      ],
  )
  def seg_kernel(table_hbm, src_hbm, dst_hbm, zeros_hbm, out_hbm,
                 src_v, dst_v, rows_v, shared, sem):
    c = lax.axis_index("c")
    s = lax.axis_index("s")
    wid = s * NC + c
    pltpu.sync_copy(zeros_hbm.at[pl.ds(s * chunk, chunk)],
                    shared.at[pl.ds(s * chunk, chunk)])

    def load_idx(mb):
      pltpu.sync_copy(src_hbm.at[wid, pl.ds(0, mb)], src_v.at[pl.ds(0, mb)])
      pltpu.sync_copy(dst_hbm.at[wid, pl.ds(0, mb)], dst_v.at[pl.ds(0, mb)])

    def run_core(mb):
      def body(j, carry):
        pltpu.async_copy(table_hbm.at[src_v.at[j]], rows_v, sem).wait()
        pltpu.sync_copy(rows_v, shared.at[dst_v.at[j]], add=True)
        return carry

      lax.fori_loop(0, mb, body, 0)

    if nb0 == nb1:
      load_idx(nb0)
      plsc.subcore_barrier()
      run_core(nb0)
    else:
      @pl.when(c == 0)
      def _():
        load_idx(nb0)

      @pl.when(c != 0)
      def _():
        load_idx(nb1)

      plsc.subcore_barrier()

      @pl.when(c == 0)
      def _():
        run_core(nb0)

      @pl.when(c != 0)
      def _():
        run_core(nb1)

    plsc.subcore_barrier()
    pltpu.sync_copy(shared.at[pl.ds(s * chunk, chunk)],
                    out_hbm.at[c, pl.ds(s * chunk, chunk)])

  return seg_kernel


def _dinv(hist_ref):
  deg = hist_ref[:, 0:1] + hist_ref[:, 1:2] + 1.0
  return lax.rsqrt(deg)


def _tc1_body(x_ref, w_ref, hist_ref, out_ref):
  h = jnp.dot(x_ref[...], w_ref[...], preferred_element_type=jnp.float32)
  out_ref[...] = h * _dinv(hist_ref)


def _tc2_body(p0_ref, p1_ref, h1_ref, hist_ref, b1_ref, w2_ref, out_ref):
  dinv = _dinv(hist_ref)
  s1 = (p0_ref[...] + p1_ref[...] + h1_ref[...]) * dinv + b1_ref[...]
  s1 = jnp.maximum(s1, 0.0)
  out_ref[...] = jnp.dot(
      s1, w2_ref[...], preferred_element_type=jnp.float32) * dinv


def _tc3_body(q0_ref, q1_ref, h2_ref, hist_ref, b2_ref, out_ref, *, d, cls):
  dinv = _dinv(hist_ref)
  o = (q0_ref[...] + q1_ref[...] + h2_ref[...]) * dinv + b2_ref[...]
  mask = lax.broadcasted_iota(jnp.int32, (R, d), 1) < cls
  om = jnp.where(mask, o, -1e30)
  m = jnp.max(om, axis=1, keepdims=True)
  e = jnp.where(mask, jnp.exp(o - m), 0.0)
  lse = jnp.log(jnp.sum(e, axis=1, keepdims=True))
  out_ref[...] = o - m - lse


def kernel(x, edge_index, W1, b1, W2, b2):
  n, f_in = x.shape
  hid = W1.shape[1]
  cls = W2.shape[1]
  e = edge_index.shape[1]

  n_pad = ((n + NS * 8 - 1) // (NS * 8)) * (NS * 8)
  if n_pad % R != 0:
    n_pad = ((n_pad + R - 1) // R) * R
  d2 = ((cls + 127) // 128) * 128

  nb = (e + NW * EB - 1) // (NW * EB)
  e_pad = NW * nb * EB
  src = jnp.concatenate(
      [edge_index[0], jnp.full((e_pad - e,), n, jnp.int32)]).reshape(
          NW, nb, EB)
  dst = jnp.concatenate(
      [edge_index[1], jnp.full((e_pad - e,), n, jnp.int32)]).reshape(
          NW, nb, EB)

  batches_needed = (e + NS * EB - 1) // (NS * EB)
  nb0 = min(max(8, 8 * round(F0 * batches_needed / 8)),
            8 * ((batches_needed + 7) // 8))
  nb1 = max(8, 8 * ((max(batches_needed - nb0, 0) + 7) // 8))
  nb_max = max(nb0, nb1)
  e0 = min(e, NS * nb0 * EB)

  def _pack(flat, count, nb_x):
    slots = NS * nb_x * EB
    a = jnp.concatenate([flat, jnp.full((slots - count,), n, jnp.int32)])
    a = a.reshape(NS, nb_x, EB)
    if nb_x < nb_max:
      a = jnp.concatenate(
          [a, jnp.full((NS, nb_max - nb_x, EB), n, jnp.int32)], axis=1)
    return a

  srcw = jnp.stack([_pack(edge_index[0][:e0], e0, nb0),
                    _pack(edge_index[0][e0:], e - e0, nb1)],
                   axis=1).reshape(NW, nb_max, EB)
  dstw = jnp.stack([_pack(edge_index[1][:e0], e0, nb0),
                    _pack(edge_index[1][e0:], e - e0, nb1)],
                   axis=1).reshape(NW, nb_max, EB)
  x_pad = jnp.pad(x, ((0, n_pad - n), (0, 0)))
  w2_pad = jnp.pad(W2, ((0, 0), (0, d2 - cls)))
  b1_2d = b1.reshape(1, hid)
  b2_2d = jnp.pad(b2, (0, d2 - cls)).reshape(1, d2)
  ones_eb = jnp.ones((EB,), jnp.float32)
  zeros_1d = jnp.zeros((n_pad,), jnp.float32)
  zeros_h = jnp.zeros((n_pad, hid), jnp.float32)
  zeros_c = jnp.zeros((n_pad, d2), jnp.float32)

  degp = _make_deg_kernel(n_pad, nb)(dst, ones_eb, zeros_1d)
  hist_t = degp.T

  grid1 = n_pad // R

  hs1 = pl.pallas_call(
      _tc1_body,
      grid=(grid1,),
      in_specs=[
          pl.BlockSpec((R, f_in), lambda j: (j, 0)),
          pl.BlockSpec((f_in, hid), lambda j: (0, 0)),
          pl.BlockSpec((R, NC), lambda j: (j, 0)),
      ],
      out_specs=pl.BlockSpec((R, hid), lambda j: (j, 0)),
      out_shape=jax.ShapeDtypeStruct((n_pad, hid), jnp.float32),
  )(x_pad, W1, hist_t)

  part1 = _make_seg_kernel(n_pad, nb_max, nb0, nb1, hid)(
      hs1, srcw, dstw, zeros_h)

  hs2 = pl.pallas_call(
      _tc2_body,
      grid=(grid1,),
      in_specs=[
          pl.BlockSpec((R, hid), lambda j: (j, 0)),
          pl.BlockSpec((R, hid), lambda j: (j, 0)),
          pl.BlockSpec((R, hid), lambda j: (j, 0)),
          pl.BlockSpec((R, NC), lambda j: (j, 0)),
          pl.BlockSpec((1, hid), lambda j: (0, 0)),
          pl.BlockSpec((hid, d2), lambda j: (0, 0)),
      ],
      out_specs=pl.BlockSpec((R, d2), lambda j: (j, 0)),
      out_shape=jax.ShapeDtypeStruct((n_pad, d2), jnp.float32),
  )(part1[0], part1[1], hs1, hist_t, b1_2d, w2_pad)

  part2 = _make_seg_kernel(n_pad, nb_max, nb0, nb1, d2)(
      hs2, srcw, dstw, zeros_c)

  out = pl.pallas_call(
      functools.partial(_tc3_body, d=d2, cls=cls),
      grid=(grid1,),
      in_specs=[
          pl.BlockSpec((R, d2), lambda j: (j, 0)),
          pl.BlockSpec((R, d2), lambda j: (j, 0)),
          pl.BlockSpec((R, d2), lambda j: (j, 0)),
          pl.BlockSpec((R, NC), lambda j: (j, 0)),
          pl.BlockSpec((1, d2), lambda j: (0, 0)),
      ],
      out_specs=pl.BlockSpec((R, d2), lambda j: (j, 0)),
      out_shape=jax.ShapeDtypeStruct((n_pad, d2), jnp.float32),
  )(part2[0], part2[1], hs2, hist_t, b2_2d)

  return out[:n, :cls]


# speedup vs baseline: 1.1553x; 1.1553x over previous
import functools

import jax
import jax.numpy as jnp
from jax import lax
from jax.experimental import pallas as pl
from jax.experimental.pallas import tpu as pltpu
from jax.experimental.pallas import tpu_sc as plsc

NC = 2
NS = 16
NW = NC * NS
EB = 128
F0 = 0.5

R = 512


def _sc_mesh():
  return plsc.VectorSubcoreMesh(
      core_axis_name="c", subcore_axis_name="s",
      num_cores=NC, num_subcores=NS)


def _make_deg_kernel(n_pad, nb):
  chunk = n_pad // NS

  @functools.partial(
      pl.kernel,
      mesh=_sc_mesh(),
      out_type=jax.ShapeDtypeStruct((NC, n_pad), jnp.float32),
      scratch_types=[
          pltpu.VMEM((nb, EB), jnp.int32),
          pltpu.VMEM((EB,), jnp.float32),
          pltpu.VMEM_SHARED((n_pad,), jnp.float32),
      ],
  )
  def deg_kernel(dst_hbm, ones_hbm, zeros_hbm, out_hbm, dst_v, ones_v, shared):
    c = lax.axis_index("c")
    s = lax.axis_index("s")
    wid = s * NC + c
    pltpu.sync_copy(zeros_hbm.at[pl.ds(s * chunk, chunk)],
                    shared.at[pl.ds(s * chunk, chunk)])
    pltpu.sync_copy(dst_hbm.at[wid], dst_v)
    pltpu.sync_copy(ones_hbm, ones_v)
    plsc.subcore_barrier()

    def body(j, carry):
      pltpu.sync_copy(ones_v, shared.at[dst_v.at[j]], add=True)
      return carry

    lax.fori_loop(0, nb, body, 0)
    plsc.subcore_barrier()
    pltpu.sync_copy(shared.at[pl.ds(s * chunk, chunk)],
                    out_hbm.at[c, pl.ds(s * chunk, chunk)])

  return deg_kernel


def _make_seg_kernel(n_pad, nb_max, nb0, nb1, d):
  chunk = n_pad // NS

  @functools.partial(
      pl.kernel,
      mesh=_sc_mesh(),
      out_type=jax.ShapeDtypeStruct((NC, n_pad, d), jnp.float32),
      scratch_types=[
          pltpu.VMEM((nb_max, EB), jnp.int32),
          pltpu.VMEM((nb_max, EB), jnp.int32),
          pltpu.VMEM((EB, d), jnp.float32),
          pltpu.VMEM_SHARED((n_pad, d), jnp.float32),
          pltpu.SemaphoreType.DMA,
      ],
  )
  def seg_kernel(table_hbm, src_hbm, dst_hbm, zeros_hbm, out_hbm,
                 src_v, dst_v, rows_v, shared, sem):
    c = lax.axis_index("c")
    s = lax.axis_index("s")
    wid = s * NC + c
    pltpu.sync_copy(zeros_hbm.at[pl.ds(s * chunk, chunk)],
                    shared.at[pl.ds(s * chunk, chunk)])

    def load_idx(mb):
      pltpu.sync_copy(src_hbm.at[wid, pl.ds(0, mb)], src_v.at[pl.ds(0, mb)])
      pltpu.sync_copy(dst_hbm.at[wid, pl.ds(0, mb)], dst_v.at[pl.ds(0, mb)])

    def run_core(mb):
      def body(j, carry):
        pltpu.async_copy(table_hbm.at[src_v.at[j]], rows_v, sem).wait()
        pltpu.sync_copy(rows_v, shared.at[dst_v.at[j]], add=True)
        return carry

      lax.fori_loop(0, mb, body, 0)

    if nb0 == nb1:
      load_idx(nb0)
      plsc.subcore_barrier()
      run_core(nb0)
    else:
      @pl.when(c == 0)
      def _():
        load_idx(nb0)

      @pl.when(c != 0)
      def _():
        load_idx(nb1)

      plsc.subcore_barrier()

      @pl.when(c == 0)
      def _():
        run_core(nb0)

      @pl.when(c != 0)
      def _():
        run_core(nb1)

    plsc.subcore_barrier()
    pltpu.sync_copy(shared.at[pl.ds(s * chunk, chunk)],
                    out_hbm.at[c, pl.ds(s * chunk, chunk)])

  return seg_kernel


def _dinv(hist_ref):
  deg = hist_ref[:, 0:1] + hist_ref[:, 1:2] + 1.0
  return lax.rsqrt(deg)


def _tc1_body(x_ref, w_ref, hist_ref, out_ref):
  h = jnp.dot(x_ref[...], w_ref[...], preferred_element_type=jnp.float32)
  out_ref[...] = h * _dinv(hist_ref)


def _tc2_body(p0_ref, p1_ref, h1_ref, hist_ref, b1_ref, w2_ref, out_ref):
  dinv = _dinv(hist_ref)
  s1 = (p0_ref[...] + p1_ref[...] + h1_ref[...]) * dinv + b1_ref[...]
  s1 = jnp.maximum(s1, 0.0)
  out_ref[...] = jnp.dot(
      s1, w2_ref[...], preferred_element_type=jnp.float32) * dinv


def _tc3_body(q0_ref, q1_ref, h2_ref, hist_ref, b2_ref, out_ref, *, d, cls):
  dinv = _dinv(hist_ref)
  o = (q0_ref[...] + q1_ref[...] + h2_ref[...]) * dinv + b2_ref[...]
  mask = lax.broadcasted_iota(jnp.int32, (R, d), 1) < cls
  om = jnp.where(mask, o, -1e30)
  m = jnp.max(om, axis=1, keepdims=True)
  e = jnp.where(mask, jnp.exp(o - m), 0.0)
  lse = jnp.log(jnp.sum(e, axis=1, keepdims=True))
  out_ref[...] = o - m - lse


def kernel(x, edge_index, W1, b1, W2, b2):
  n, f_in = x.shape
  hid = W1.shape[1]
  cls = W2.shape[1]
  e = edge_index.shape[1]

  n_pad = ((n + NS * 8 - 1) // (NS * 8)) * (NS * 8)
  if n_pad % R != 0:
    n_pad = ((n_pad + R - 1) // R) * R
  d2 = ((cls + 127) // 128) * 128

  nb = (e + NW * EB - 1) // (NW * EB)
  e_pad = NW * nb * EB
  src = jnp.concatenate(
      [edge_index[0], jnp.full((e_pad - e,), n, jnp.int32)]).reshape(
          NW, nb, EB)
  dst = jnp.concatenate(
      [edge_index[1], jnp.full((e_pad - e,), n, jnp.int32)]).reshape(
          NW, nb, EB)

  batches_needed = (e + NS * EB - 1) // (NS * EB)
  nb0 = min(max(8, 8 * round(F0 * batches_needed / 8)),
            8 * ((batches_needed + 7) // 8))
  nb1 = max(8, 8 * ((max(batches_needed - nb0, 0) + 7) // 8))
  nb_max = max(nb0, nb1)
  e0 = min(e, NS * nb0 * EB)

  def _pack(flat, count, nb_x):
    slots = NS * nb_x * EB
    a = jnp.concatenate([flat, jnp.full((slots - count,), n, jnp.int32)])
    a = a.reshape(NS, nb_x, EB)
    if nb_x < nb_max:
      a = jnp.concatenate(
          [a, jnp.full((NS, nb_max - nb_x, EB), n, jnp.int32)], axis=1)
    return a

  srcw = jnp.stack([_pack(edge_index[0][:e0], e0, nb0),
                    _pack(edge_index[0][e0:], e - e0, nb1)],
                   axis=1).reshape(NW, nb_max, EB)
  dstw = jnp.stack([_pack(edge_index[1][:e0], e0, nb0),
                    _pack(edge_index[1][e0:], e - e0, nb1)],
                   axis=1).reshape(NW, nb_max, EB)
  x_pad = jnp.pad(x, ((0, n_pad - n), (0, 0)))
  w2_pad = jnp.pad(W2, ((0, 0), (0, d2 - cls)))
  b1_2d = b1.reshape(1, hid)
  b2_2d = jnp.pad(b2, (0, d2 - cls)).reshape(1, d2)
  ones_eb = jnp.ones((EB,), jnp.float32)
  zeros_1d = jnp.zeros((n_pad,), jnp.float32)
  zeros_h = jnp.zeros((n_pad, hid), jnp.float32)
  zeros_c = jnp.zeros((n_pad, d2), jnp.float32)

  degp = _make_deg_kernel(n_pad, nb)(dst, ones_eb, zeros_1d)
  hist_t = degp.T

  grid1 = n_pad // R

  hs1 = pl.pallas_call(
      _tc1_body,
      grid=(grid1,),
      in_specs=[
          pl.BlockSpec((R, f_in), lambda j: (j, 0)),
          pl.BlockSpec((f_in, hid), lambda j: (0, 0)),
          pl.BlockSpec((R, NC), lambda j: (j, 0)),
      ],
      out_specs=pl.BlockSpec((R, hid), lambda j: (j, 0)),
      out_shape=jax.ShapeDtypeStruct((n_pad, hid), jnp.float32),
  )(x_pad, W1, hist_t)

  part1 = _make_seg_kernel(n_pad, nb_max, nb0, nb1, hid)(
      hs1, srcw, dstw, zeros_h)

  hs2 = pl.pallas_call(
      _tc2_body,
      grid=(grid1,),
      in_specs=[
          pl.BlockSpec((R, hid), lambda j: (j, 0)),
          pl.BlockSpec((R, hid), lambda j: (j, 0)),
          pl.BlockSpec((R, hid), lambda j: (j, 0)),
          pl.BlockSpec((R, NC), lambda j: (j, 0)),
          pl.BlockSpec((1, hid), lambda j: (0, 0)),
          pl.BlockSpec((hid, d2), lambda j: (0, 0)),
      ],
      out_specs=pl.BlockSpec((R, d2), lambda j: (j, 0)),
      out_shape=jax.ShapeDtypeStruct((n_pad, d2), jnp.float32),
  )(part1[0], part1[1], hs1, hist_t, b1_2d, w2_pad)

  part2 = _make_seg_kernel(n_pad, nb_max, nb0, nb1, d2)(
      hs2, srcw, dstw, zeros_c)

  out = pl.pallas_call(
      functools.partial(_tc3_body, d=d2, cls=cls),
      grid=(grid1,),
      in_specs=[
          pl.BlockSpec((R, d2), lambda j: (j, 0)),
          pl.BlockSpec((R, d2), lambda j: (j, 0)),
          pl.BlockSpec((R, d2), lambda j: (j, 0)),
          pl.BlockSpec((R, NC), lambda j: (j, 0)),
          pl.BlockSpec((1, d2), lambda j: (0, 0)),
      ],
      out_specs=pl.BlockSpec((R, d2), lambda j: (j, 0)),
      out_shape=jax.ShapeDtypeStruct((n_pad, d2), jnp.float32),
  )(part2[0], part2[1], hs2, hist_t, b2_2d)

  return out[:n, :cls]


# speedup vs baseline: 2.6280x; 2.2747x over previous
import functools

import jax
import jax.numpy as jnp
from jax import lax
from jax.experimental import pallas as pl
from jax.experimental.pallas import tpu as pltpu
from jax.experimental.pallas import tpu_sc as plsc

NC = 2
NS = 16
NW = NC * NS
EB = 128
F0 = 0.5

R = 512


def _sc_mesh():
  return plsc.VectorSubcoreMesh(
      core_axis_name="c", subcore_axis_name="s",
      num_cores=NC, num_subcores=NS)


def _make_deg_kernel(n_pad, nb):
  chunk = n_pad // NS

  @functools.partial(
      pl.kernel,
      mesh=_sc_mesh(),
      out_type=jax.ShapeDtypeStruct((NC, n_pad), jnp.float32),
      scratch_types=[
          pltpu.VMEM((nb, EB), jnp.int32),
          pltpu.VMEM((EB,), jnp.float32),
          pltpu.VMEM_SHARED((n_pad,), jnp.float32),
      ],
  )
  def deg_kernel(dst_hbm, ones_hbm, zeros_hbm, out_hbm, dst_v, ones_v, shared):
    c = lax.axis_index("c")
    s = lax.axis_index("s")
    wid = s * NC + c
    pltpu.sync_copy(zeros_hbm.at[pl.ds(s * chunk, chunk)],
                    shared.at[pl.ds(s * chunk, chunk)])
    pltpu.sync_copy(dst_hbm.at[wid], dst_v)
    pltpu.sync_copy(ones_hbm, ones_v)
    plsc.subcore_barrier()

    def body(j, carry):
      pltpu.sync_copy(ones_v, shared.at[dst_v.at[j]], add=True)
      return carry

    lax.fori_loop(0, nb, body, 0)
    plsc.subcore_barrier()
    pltpu.sync_copy(shared.at[pl.ds(s * chunk, chunk)],
                    out_hbm.at[c, pl.ds(s * chunk, chunk)])

  return deg_kernel


def _make_seg_kernel(n_pad, nb_max, nb0, nb1, d):
  chunk = n_pad // NS

  @functools.partial(
      pl.kernel,
      mesh=_sc_mesh(),
      out_type=jax.ShapeDtypeStruct((NC, n_pad, d), jnp.float32),
      scratch_types=[
          pltpu.VMEM((nb_max, EB), jnp.int32),
          pltpu.VMEM((nb_max, EB), jnp.int32),
          pltpu.VMEM((EB, d), jnp.float32),
          pltpu.VMEM_SHARED((n_pad, d), jnp.float32),
          pltpu.SemaphoreType.DMA,
      ],
  )
  def seg_kernel(table_hbm, src_hbm, dst_hbm, zeros_hbm, out_hbm,
                 src_v, dst_v, rows_v, shared, sem):
    c = lax.axis_index("c")
    s = lax.axis_index("s")
    wid = s * NC + c
    pltpu.sync_copy(zeros_hbm.at[pl.ds(s * chunk, chunk)],
                    shared.at[pl.ds(s * chunk, chunk)])

    def load_idx(mb):
      pltpu.sync_copy(src_hbm.at[wid, pl.ds(0, mb)], src_v.at[pl.ds(0, mb)])
      pltpu.sync_copy(dst_hbm.at[wid, pl.ds(0, mb)], dst_v.at[pl.ds(0, mb)])

    def run_core(mb):
      def body(j, carry):
        pltpu.async_copy(table_hbm.at[src_v.at[j]], rows_v, sem).wait()
        pltpu.sync_copy(rows_v, shared.at[dst_v.at[j]], add=True)
        return carry

      lax.fori_loop(0, mb, body, 0)

    if nb0 == nb1:
      load_idx(nb0)
      plsc.subcore_barrier()
      run_core(nb0)
    else:
      @pl.when(c == 0)
      def _():
        load_idx(nb0)

      @pl.when(c != 0)
      def _():
        load_idx(nb1)

      plsc.subcore_barrier()

      @pl.when(c == 0)
      def _():
        run_core(nb0)

      @pl.when(c != 0)
      def _():
        run_core(nb1)

    plsc.subcore_barrier()
    pltpu.sync_copy(shared.at[pl.ds(s * chunk, chunk)],
                    out_hbm.at[c, pl.ds(s * chunk, chunk)])

  return seg_kernel


def _dinv(hist_ref):
  deg = hist_ref[:, 0:1] + hist_ref[:, 1:2] + 1.0
  return lax.rsqrt(deg)


def _tc1_body(x_ref, w_ref, hist_ref, out_ref):
  h = jnp.dot(x_ref[...], w_ref[...], preferred_element_type=jnp.float32)
  out_ref[...] = h * _dinv(hist_ref)


def _tc2_body(p0_ref, p1_ref, h1_ref, hist_ref, b1_ref, w2_ref, out_ref):
  dinv = _dinv(hist_ref)
  s1 = (p0_ref[...] + p1_ref[...] + h1_ref[...]) * dinv + b1_ref[...]
  s1 = jnp.maximum(s1, 0.0)
  out_ref[...] = jnp.dot(
      s1, w2_ref[...], preferred_element_type=jnp.float32) * dinv


def _tc3_body(q0_ref, q1_ref, h2_ref, hist_ref, b2_ref, out_ref, *, d, cls):
  dinv = _dinv(hist_ref)
  o = (q0_ref[...] + q1_ref[...] + h2_ref[...]) * dinv + b2_ref[...]
  mask = lax.broadcasted_iota(jnp.int32, (R, d), 1) < cls
  om = jnp.where(mask, o, -1e30)
  m = jnp.max(om, axis=1, keepdims=True)
  e = jnp.where(mask, jnp.exp(o - m), 0.0)
  lse = jnp.log(jnp.sum(e, axis=1, keepdims=True))
  out_ref[...] = o - m - lse


def kernel(x, edge_index, W1, b1, W2, b2):
  n, f_in = x.shape
  hid = W1.shape[1]
  cls = W2.shape[1]
  e = edge_index.shape[1]

  n_pad = ((n + NS * 8 - 1) // (NS * 8)) * (NS * 8)
  if n_pad % R != 0:
    n_pad = ((n_pad + R - 1) // R) * R
  d2 = ((cls + 127) // 128) * 128

  def _dummies(count):
    return n + (jnp.arange(count, dtype=jnp.int32) % (n_pad - n))

  nb = (e + NW * EB - 1) // (NW * EB)
  e_pad = NW * nb * EB
  src = jnp.concatenate(
      [edge_index[0], _dummies(e_pad - e)]).reshape(NW, nb, EB)
  dst = jnp.concatenate(
      [edge_index[1], _dummies(e_pad - e)]).reshape(NW, nb, EB)

  batches_needed = (e + NS * EB - 1) // (NS * EB)
  nb0 = min(max(8, 8 * round(F0 * batches_needed / 8)),
            8 * ((batches_needed + 7) // 8))
  nb1 = max(8, 8 * ((max(batches_needed - nb0, 0) + 7) // 8))
  nb_max = max(nb0, nb1)
  e0 = min(e, NS * nb0 * EB)

  def _pack(flat, count, nb_x):
    slots = NS * nb_x * EB
    a = jnp.concatenate([flat, _dummies(slots - count)])
    a = a.reshape(NS, nb_x, EB)
    if nb_x < nb_max:
      a = jnp.concatenate(
          [a, _dummies(NS * (nb_max - nb_x) * EB).reshape(
              NS, nb_max - nb_x, EB)], axis=1)
    return a

  srcw = jnp.stack([_pack(edge_index[0][:e0], e0, nb0),
                    _pack(edge_index[0][e0:], e - e0, nb1)],
                   axis=1).reshape(NW, nb_max, EB)
  dstw = jnp.stack([_pack(edge_index[1][:e0], e0, nb0),
                    _pack(edge_index[1][e0:], e - e0, nb1)],
                   axis=1).reshape(NW, nb_max, EB)
  x_pad = jnp.pad(x, ((0, n_pad - n), (0, 0)))
  w2_pad = jnp.pad(W2, ((0, 0), (0, d2 - cls)))
  b1_2d = b1.reshape(1, hid)
  b2_2d = jnp.pad(b2, (0, d2 - cls)).reshape(1, d2)
  ones_eb = jnp.ones((EB,), jnp.float32)
  zeros_1d = jnp.zeros((n_pad,), jnp.float32)
  zeros_h = jnp.zeros((n_pad, hid), jnp.float32)
  zeros_c = jnp.zeros((n_pad, d2), jnp.float32)

  degp = _make_deg_kernel(n_pad, nb)(dst, ones_eb, zeros_1d)
  hist_t = degp.T

  grid1 = n_pad // R

  hs1 = pl.pallas_call(
      _tc1_body,
      grid=(grid1,),
      in_specs=[
          pl.BlockSpec((R, f_in), lambda j: (j, 0)),
          pl.BlockSpec((f_in, hid), lambda j: (0, 0)),
          pl.BlockSpec((R, NC), lambda j: (j, 0)),
      ],
      out_specs=pl.BlockSpec((R, hid), lambda j: (j, 0)),
      out_shape=jax.ShapeDtypeStruct((n_pad, hid), jnp.float32),
  )(x_pad, W1, hist_t)

  part1 = _make_seg_kernel(n_pad, nb_max, nb0, nb1, hid)(
      hs1, srcw, dstw, zeros_h)

  hs2 = pl.pallas_call(
      _tc2_body,
      grid=(grid1,),
      in_specs=[
          pl.BlockSpec((R, hid), lambda j: (j, 0)),
          pl.BlockSpec((R, hid), lambda j: (j, 0)),
          pl.BlockSpec((R, hid), lambda j: (j, 0)),
          pl.BlockSpec((R, NC), lambda j: (j, 0)),
          pl.BlockSpec((1, hid), lambda j: (0, 0)),
          pl.BlockSpec((hid, d2), lambda j: (0, 0)),
      ],
      out_specs=pl.BlockSpec((R, d2), lambda j: (j, 0)),
      out_shape=jax.ShapeDtypeStruct((n_pad, d2), jnp.float32),
  )(part1[0], part1[1], hs1, hist_t, b1_2d, w2_pad)

  part2 = _make_seg_kernel(n_pad, nb_max, nb0, nb1, d2)(
      hs2, srcw, dstw, zeros_c)

  out = pl.pallas_call(
      functools.partial(_tc3_body, d=d2, cls=cls),
      grid=(grid1,),
      in_specs=[
          pl.BlockSpec((R, d2), lambda j: (j, 0)),
          pl.BlockSpec((R, d2), lambda j: (j, 0)),
          pl.BlockSpec((R, d2), lambda j: (j, 0)),
          pl.BlockSpec((R, NC), lambda j: (j, 0)),
          pl.BlockSpec((1, d2), lambda j: (0, 0)),
      ],
      out_specs=pl.BlockSpec((R, d2), lambda j: (j, 0)),
      out_shape=jax.ShapeDtypeStruct((n_pad, d2), jnp.float32),
  )(part2[0], part2[1], hs2, hist_t, b2_2d)

  return out[:n, :cls]
